# baseline (speedup 1.0000x reference)
import functools

import jax
import jax.numpy as jnp
from jax.experimental import pallas as pl
from jax.experimental.pallas import tpu as pltpu

MAX_WIDTH = 0.1
N_MASKS = 2
MASK_VALUE = 0.0

_BT = 512


def _mask_kernel(starts_ref, ends_ref, x_ref, o_ref):
    b = pl.program_id(0)
    tb = pl.program_id(1)
    t0 = tb * _BT
    rows = t0 + jax.lax.broadcasted_iota(jnp.int32, (1, _BT, 1), 1)
    masked = jnp.zeros(rows.shape, dtype=jnp.bool_)
    for m in range(N_MASKS):
        s = starts_ref[b, m]
        e = ends_ref[b, m]
        masked = masked | ((rows >= s) & (rows < e))
    o_ref[...] = jnp.where(masked, jnp.float32(MASK_VALUE), x_ref[...])


def _spans(B, T):
    kw, ks = jax.random.split(jax.random.key(1))
    max_w = int(MAX_WIDTH * T)
    widths = jax.random.randint(kw, (B, N_MASKS), 1, max_w + 1)
    starts = jax.random.randint(ks, (B, N_MASKS), 0, T)
    starts = jnp.minimum(starts, T - widths)
    return starts.astype(jnp.int32), (starts + widths).astype(jnp.int32)


@jax.jit
def kernel(x):
    B, T, F = x.shape
    starts, ends = _spans(B, T)
    grid = (B, T // _BT)
    return pl.pallas_call(
        _mask_kernel,
        grid_spec=pltpu.PrefetchScalarGridSpec(
            num_scalar_prefetch=2,
            grid=grid,
            in_specs=[
                pl.BlockSpec((1, _BT, F), lambda b, t, s0, s1: (b, t, 0)),
            ],
            out_specs=pl.BlockSpec((1, _BT, F), lambda b, t, s0, s1: (b, t, 0)),
        ),
        out_shape=jax.ShapeDtypeStruct(x.shape, x.dtype),
    )(starts, ends, x)


# speedup vs baseline: 1.0112x; 1.0112x over previous
import functools

import jax
import jax.numpy as jnp
from jax.experimental import pallas as pl
from jax.experimental.pallas import tpu as pltpu

MAX_WIDTH = 0.1
N_MASKS = 2
MASK_VALUE = 0.0

_BT = 1024


def _mask_kernel(starts_ref, ends_ref, x_ref, o_ref):
    b = pl.program_id(0)
    tb = pl.program_id(1)
    t0 = tb * _BT
    rows = t0 + jax.lax.broadcasted_iota(jnp.int32, (1, _BT, 1), 1)
    masked = jnp.zeros(rows.shape, dtype=jnp.bool_)
    for m in range(N_MASKS):
        s = starts_ref[b, m]
        e = ends_ref[b, m]
        masked = masked | ((rows >= s) & (rows < e))
    o_ref[...] = jnp.where(masked, jnp.float32(MASK_VALUE), x_ref[...])


def _spans(B, T):
    kw, ks = jax.random.split(jax.random.key(1))
    max_w = int(MAX_WIDTH * T)
    widths = jax.random.randint(kw, (B, N_MASKS), 1, max_w + 1)
    starts = jax.random.randint(ks, (B, N_MASKS), 0, T)
    starts = jnp.minimum(starts, T - widths)
    return starts.astype(jnp.int32), (starts + widths).astype(jnp.int32)


@jax.jit
def kernel(x):
    B, T, F = x.shape
    starts, ends = _spans(B, T)
    grid = (B, T // _BT)
    return pl.pallas_call(
        _mask_kernel,
        grid_spec=pltpu.PrefetchScalarGridSpec(
            num_scalar_prefetch=2,
            grid=grid,
            in_specs=[
                pl.BlockSpec((1, _BT, F), lambda b, t, s0, s1: (b, t, 0)),
            ],
            out_specs=pl.BlockSpec((1, _BT, F), lambda b, t, s0, s1: (b, t, 0)),
        ),
        out_shape=jax.ShapeDtypeStruct(x.shape, x.dtype),
    )(starts, ends, x)
